# baseline (speedup 1.0000x reference)
import functools

import numpy as np
import jax
import jax.numpy as jnp
from jax.experimental import pallas as pl
from jax.experimental.pallas import tpu as pltpu

_K = 5
_M = 10
_NV = 100
_TOL = 1e-05

_N = 5000
_D = 128
_BLK = 200


def _topk_body(x_ref, xt_ref, dd_ref, inds_ref):
    i = pl.program_id(0)
    G = jax.lax.dot_general(
        x_ref[...], xt_ref[...], (((1,), (0,)), ((), ())),
        preferred_element_type=jnp.float32)
    val = 2.0 * G - dd_ref[0:1, :]
    iota = jax.lax.broadcasted_iota(jnp.int32, val.shape, 1)
    rid = i * _BLK + jax.lax.broadcasted_iota(jnp.int32, (_BLK, 1), 0)
    for t in range(8):
        if t < _K + 1:
            m = jnp.max(val, axis=1, keepdims=True)
            idx = jnp.min(jnp.where(val == m, iota, jnp.int32(2**30)),
                          axis=1, keepdims=True)
            inds_ref[:, t:t + 1] = idx
            val = jnp.where(iota == idx, -jnp.inf, val)
        else:
            inds_ref[:, t:t + 1] = rid


def _knn_inds(x):
    n, d = x.shape
    dd = jnp.sum(x * x, axis=1)
    dd8 = jnp.broadcast_to(dd[None, :], (8, n))
    return pl.pallas_call(
        _topk_body,
        grid=(n // _BLK,),
        in_specs=[pl.BlockSpec((_BLK, d), lambda i: (i, 0)),
                  pl.BlockSpec((d, n), lambda i: (0, 0)),
                  pl.BlockSpec((8, n), lambda i: (0, 0))],
        out_specs=pl.BlockSpec((_BLK, 8), lambda i: (i, 0)),
        out_shape=jax.ShapeDtypeStruct((n, 8), jnp.int32),
    )(x, x.T, dd8)


def _sparse_graph(x, k):
    n = x.shape[0]
    inds = _knn_inds(x)
    j = inds[:, :k + 1]
    i = jnp.arange(n)
    wf = j != i[:, None]
    nbr = inds[j]
    mutual = jnp.any(nbr == i[:, None, None], axis=-1)
    wb = wf & jnp.logical_not(mutual)
    deg = wf.sum(1).astype(jnp.float32)
    deg = deg + jnp.zeros((n,), jnp.float32).at[j.reshape(-1)].add(
        wb.reshape(-1).astype(jnp.float32))
    dsq = 1.0 / jnp.sqrt(deg)
    gsrc = jnp.where(wf, j, n)
    sdst = jnp.where(wb, j, n + 1)
    return gsrc, sdst, dsq


def _matvec(gsrc, sdst, dsq, S):
    n, nv = S.shape
    Y = dsq[:, None] * S
    Yp = jnp.concatenate([Y, jnp.zeros((1, nv), jnp.float32)], axis=0)
    acc = Yp[gsrc[:, 0]]
    for a in range(1, gsrc.shape[1]):
        acc = acc + Yp[gsrc[:, a]]
    sc = jnp.zeros((n + 2, nv), jnp.float32)
    for a in range(sdst.shape[1]):
        sc = sc.at[sdst[:, a]].add(Y)
    P = acc + sc[:n]
    return S - dsq[:, None] * P


def _lanczos(G, m, nv, key):
    gsrc, sdst, dsq = G
    n = dsq.shape[0]
    SV = jax.random.normal(key, (n, nv), dtype=jnp.float32)
    SV = SV / jnp.linalg.norm(SV, axis=0)
    V = jnp.zeros((n, m, nv), dtype=jnp.float32)
    T = jnp.zeros((nv, m, m), dtype=jnp.float32)
    V = V.at[:, 0, :].set(SV)
    w = _matvec(gsrc, sdst, dsq, SV)
    alpha = jnp.einsum('ij,ij->j', w, SV)
    w = w - alpha[None, :] * SV
    beta = jnp.sqrt(jnp.einsum('ij,ij->j', w, w))
    T = T.at[:, 0, 0].set(alpha)
    T = T.at[:, 0, 1].set(beta)
    T = T.at[:, 1, 0].set(beta)
    w = w / beta[None, :]
    V = V.at[:, 1, :].set(w)
    done = jnp.array(False)
    for i in range(1, m):
        SVold = V[:, i - 1, :]
        SVi = V[:, i, :]
        w = _matvec(gsrc, sdst, dsq, SVi)
        w = w - beta[None, :] * SVold
        alpha = jnp.einsum('ij,ij->j', w, SVi)
        Tc = T.at[:, i, i].set(alpha)
        if i < m - 1:
            w = w - alpha[None, :] * SVi
            t = jnp.einsum('ijk,ik->jk', V, w)
            w = w - jnp.einsum('ijk,jk->ik', V, t)
            beta_new = jnp.sqrt(jnp.einsum('ij,ij->j', w, w))
            w = w / beta_new[None, :]
            Tc = Tc.at[:, i, i + 1].set(beta_new)
            Tc = Tc.at[:, i + 1, i].set(beta_new)
            innerprod = jnp.einsum('ijk,ik->jk', V, w)

            def cond_fn(carry):
                cnt, w_c, ip_c = carry
                return jnp.logical_and(cnt < 100, (ip_c > _TOL).sum() > 0)

            def body_fn(carry):
                cnt, w_c, ip_c = carry
                t_c = jnp.einsum('ijk,ik->jk', V, w_c)
                w_c = w_c - jnp.einsum('ijk,jk->ik', V, t_c)
                w_c = w_c / jnp.linalg.norm(w_c, axis=0)[None, :]
                ip_c = jnp.einsum('ijk,ik->jk', V, w_c)
                return (cnt + 1, w_c, ip_c)

            cnt, w, innerprod = jax.lax.while_loop(
                cond_fn, body_fn, (jnp.int32(0), w, innerprod))
            reortho = cnt < 100
            Vc = V.at[:, i + 1, :].set(w)
            T = jnp.where(done, T, Tc)
            V = jnp.where(done, V, Vc)
            beta = jnp.where(done, beta, beta_new)
            break_cond = jnp.logical_or(
                (jnp.abs(beta_new) > 1e-06).sum() == 0,
                jnp.logical_not(reortho))
            done = jnp.logical_or(done, break_cond)
        else:
            T = jnp.where(done, T, Tc)
    return T, V


def _slq(G, n, m, niters, ts, key):
    T, _ = _lanczos(G, m, niters, key)
    eigvals, eigvecs = jnp.linalg.eigh(T)
    sqeigv1 = eigvecs[:, 0, :] ** 2
    traces = []
    for f in (jnp.exp, lambda v: v):
        expeig = f(-jnp.outer(ts, eigvals.reshape(-1))).reshape(
            ts.shape[0], niters, m)
        traces.append(n * (expeig * sqeigv1).sum(-1).mean(-1))
    subee = traces[0] - traces[1] / jnp.exp(ts)
    sub = -ts * n / jnp.exp(ts)
    return subee + sub


def _descriptor(x, ts, key):
    n = x.shape[0]
    G = _sparse_graph(x, _K)
    msid = _slq(G, n, _M, _NV, ts, key)
    return msid / n


def kernel(x_features, y_features):
    ts = jnp.asarray(np.logspace(-1, 1, 256), dtype=jnp.float32)
    mx = _descriptor(x_features, ts, jax.random.key(1))
    my = _descriptor(y_features, ts, jax.random.key(2))
    c = jnp.exp(-2.0 * (ts + 1.0 / ts))
    return jnp.amax(c * jnp.abs(mx - my))


# speedup vs baseline: 1.8324x; 1.8324x over previous
import functools

import numpy as np
import jax
import jax.numpy as jnp
from jax import lax
from jax.experimental import pallas as pl
from jax.experimental.pallas import tpu as pltpu
from jax.experimental.pallas import tpu_sc as plsc

_K = 5
_M = 10
_NV = 100
_TOL = 1e-05

_N = 5000
_D = 128
_BLK = 200


def _topk_body(x_ref, xt_ref, dd_ref, inds_ref):
    i = pl.program_id(0)
    G = jax.lax.dot_general(
        x_ref[...], xt_ref[...], (((1,), (0,)), ((), ())),
        preferred_element_type=jnp.float32)
    val = 2.0 * G - dd_ref[0:1, :]
    iota = jax.lax.broadcasted_iota(jnp.int32, val.shape, 1)
    rid = i * _BLK + jax.lax.broadcasted_iota(jnp.int32, (_BLK, 1), 0)
    for t in range(8):
        if t < _K + 1:
            m = jnp.max(val, axis=1, keepdims=True)
            idx = jnp.min(jnp.where(val == m, iota, jnp.int32(2**30)),
                          axis=1, keepdims=True)
            inds_ref[:, t:t + 1] = idx
            val = jnp.where(iota == idx, -jnp.inf, val)
        else:
            inds_ref[:, t:t + 1] = rid


def _knn_inds(x):
    n, d = x.shape
    dd = jnp.sum(x * x, axis=1)
    dd8 = jnp.broadcast_to(dd[None, :], (8, n))
    return pl.pallas_call(
        _topk_body,
        grid=(n // _BLK,),
        in_specs=[pl.BlockSpec((_BLK, d), lambda i: (i, 0)),
                  pl.BlockSpec((d, n), lambda i: (0, 0)),
                  pl.BlockSpec((8, n), lambda i: (0, 0))],
        out_specs=pl.BlockSpec((_BLK, 8), lambda i: (i, 0)),
        out_shape=jax.ShapeDtypeStruct((n, 8), jnp.int32),
    )(x, x.T, dd8)


def _sparse_graph(x, k):
    n = x.shape[0]
    inds = _knn_inds(x)
    j = inds[:, :k + 1]
    i = jnp.arange(n)
    wf = j != i[:, None]
    nbr = inds[j]
    mutual = jnp.any(nbr == i[:, None, None], axis=-1)
    wb = wf & jnp.logical_not(mutual)
    deg = wf.sum(1).astype(jnp.float32)
    deg = deg + jnp.zeros((n,), jnp.float32).at[j.reshape(-1)].add(
        wb.reshape(-1).astype(jnp.float32))
    dsq = 1.0 / jnp.sqrt(deg)
    gsrc = jnp.where(wf, j, n)
    trash = n + 1 + (i % (_NP - n - 2))
    sdst = jnp.where(wb, j, trash[:, None])
    npad = _NP - n
    gsrc_p = jnp.concatenate(
        [gsrc, jnp.full((npad, _NE), n, jnp.int32)], axis=0)
    sdst_p = jnp.concatenate(
        [sdst, jnp.full((npad, _NE), n + 1, jnp.int32)], axis=0)
    gsrc3 = gsrc_p.reshape(_NW, _R, _NE).transpose(0, 2, 1)
    sdst3 = sdst_p.reshape(_NW, _R, _NE).transpose(0, 2, 1)
    zeros = jnp.zeros((_R, _W), jnp.float32)
    return gsrc3, sdst3, zeros, dsq


_NP = 5120
_W = 128
_R = 160
_NE = _K + 1
_ZROW = _N
_NW = 32


def _sc_spmv_body(yp, gsrc3, sdst3, zeros, out,
                  acc, ychunk, zbuf, buf0, buf1,
                  idxf, idxb, idq, sem):
    c = lax.axis_index("c")
    s = lax.axis_index("s")
    wid = c * 16 + s
    base = wid * _R
    other = (1 - c) * (_NP // 2) + s * _R

    cps = [pltpu.async_copy(yp.at[pl.ds(base, _R)], ychunk, sem),
           pltpu.async_copy(zeros.at[pl.ds(0, _R)], zbuf, sem)]
    for a in range(_NE):
        cps.append(pltpu.async_copy(gsrc3.at[wid, a], idxf[a], sem))
        cps.append(pltpu.async_copy(sdst3.at[wid, a], idxb[a], sem))
    for cp in cps:
        cp.wait()

    for t in range(_R // 16):
        idq[pl.ds(t * 16, 16)] = lax.iota(jnp.int32, 16) + (base + t * 16)

    pltpu.sync_copy(zbuf, acc.at[pl.ds(other, _R)])
    pltpu.async_copy(yp.at[idxf[0]], buf0, sem).wait()
    pltpu.sync_copy(buf0, acc.at[pl.ds(base, _R)])
    plsc.subcore_barrier()

    for a in range(1, _NE):
        buf = buf0 if a % 2 == 0 else buf1
        pltpu.async_copy(yp.at[idxf[a]], buf, sem).wait()
        pltpu.sync_copy(buf, acc.at[idq], add=True)

    for a in range(_NE):
        pltpu.sync_copy(ychunk, acc.at[idxb[a]], add=True)

    plsc.subcore_barrier()

    for t2 in range(2):
        lo = s * (2 * _R) + t2 * _R
        pltpu.sync_copy(acc.at[pl.ds(lo, _R)], buf0)
        pltpu.sync_copy(buf0, out.at[c, pl.ds(lo, _R)])


@functools.partial(
    pl.kernel,
    mesh=plsc.VectorSubcoreMesh(core_axis_name="c", subcore_axis_name="s"),
    out_type=jax.ShapeDtypeStruct((2, _NP, _W), jnp.float32),
    scratch_types=[
        pltpu.VMEM_SHARED((_NP, _W), jnp.float32),
        pltpu.VMEM((_R, _W), jnp.float32),
        pltpu.VMEM((_R, _W), jnp.float32),
        pltpu.VMEM((_R, _W), jnp.float32),
        pltpu.VMEM((_R, _W), jnp.float32),
        [pltpu.VMEM((_R,), jnp.int32)] * _NE,
        [pltpu.VMEM((_R,), jnp.int32)] * _NE,
        pltpu.VMEM((_R,), jnp.int32),
        pltpu.SemaphoreType.DMA,
    ],
)
def _sc_spmv(yp, gsrc3, sdst3, zeros, out,
             acc, ychunk, zbuf, buf0, buf1, idxf, idxb, idq, sem):
    _sc_spmv_body(yp, gsrc3, sdst3, zeros, out,
                  acc, ychunk, zbuf, buf0, buf1, idxf, idxb, idq, sem)


def _matvec(gsrc3, sdst3, zeros, dsq, S):
    n, nv = S.shape
    Y = dsq[:, None] * S
    Yp = jnp.zeros((_NP, _W), jnp.float32).at[:n, :nv].set(Y)
    out = _sc_spmv(Yp, gsrc3, sdst3, zeros)
    P = out[0, :n, :nv] + out[1, :n, :nv]
    return S - dsq[:, None] * P


def _lanczos(G, m, nv, key):
    gsrc3, sdst3, zeros, dsq = G
    n = dsq.shape[0]
    SV = jax.random.normal(key, (n, nv), dtype=jnp.float32)
    SV = SV / jnp.linalg.norm(SV, axis=0)
    V = jnp.zeros((n, m, nv), dtype=jnp.float32)
    T = jnp.zeros((nv, m, m), dtype=jnp.float32)
    V = V.at[:, 0, :].set(SV)
    w = _matvec(gsrc3, sdst3, zeros, dsq, SV)
    alpha = jnp.einsum('ij,ij->j', w, SV)
    w = w - alpha[None, :] * SV
    beta = jnp.sqrt(jnp.einsum('ij,ij->j', w, w))
    T = T.at[:, 0, 0].set(alpha)
    T = T.at[:, 0, 1].set(beta)
    T = T.at[:, 1, 0].set(beta)
    w = w / beta[None, :]
    V = V.at[:, 1, :].set(w)
    done = jnp.array(False)
    for i in range(1, m):
        SVold = V[:, i - 1, :]
        SVi = V[:, i, :]
        w = _matvec(gsrc3, sdst3, zeros, dsq, SVi)
        w = w - beta[None, :] * SVold
        alpha = jnp.einsum('ij,ij->j', w, SVi)
        Tc = T.at[:, i, i].set(alpha)
        if i < m - 1:
            w = w - alpha[None, :] * SVi
            t = jnp.einsum('ijk,ik->jk', V, w)
            w = w - jnp.einsum('ijk,jk->ik', V, t)
            beta_new = jnp.sqrt(jnp.einsum('ij,ij->j', w, w))
            w = w / beta_new[None, :]
            Tc = Tc.at[:, i, i + 1].set(beta_new)
            Tc = Tc.at[:, i + 1, i].set(beta_new)
            innerprod = jnp.einsum('ijk,ik->jk', V, w)

            def cond_fn(carry):
                cnt, w_c, ip_c = carry
                return jnp.logical_and(cnt < 100, (ip_c > _TOL).sum() > 0)

            def body_fn(carry):
                cnt, w_c, ip_c = carry
                t_c = jnp.einsum('ijk,ik->jk', V, w_c)
                w_c = w_c - jnp.einsum('ijk,jk->ik', V, t_c)
                w_c = w_c / jnp.linalg.norm(w_c, axis=0)[None, :]
                ip_c = jnp.einsum('ijk,ik->jk', V, w_c)
                return (cnt + 1, w_c, ip_c)

            cnt, w, innerprod = jax.lax.while_loop(
                cond_fn, body_fn, (jnp.int32(0), w, innerprod))
            reortho = cnt < 100
            Vc = V.at[:, i + 1, :].set(w)
            T = jnp.where(done, T, Tc)
            V = jnp.where(done, V, Vc)
            beta = jnp.where(done, beta, beta_new)
            break_cond = jnp.logical_or(
                (jnp.abs(beta_new) > 1e-06).sum() == 0,
                jnp.logical_not(reortho))
            done = jnp.logical_or(done, break_cond)
        else:
            T = jnp.where(done, T, Tc)
    return T, V


def _slq(G, n, m, niters, ts, key):
    T, _ = _lanczos(G, m, niters, key)
    eigvals, eigvecs = jnp.linalg.eigh(T)
    sqeigv1 = eigvecs[:, 0, :] ** 2
    traces = []
    for f in (jnp.exp, lambda v: v):
        expeig = f(-jnp.outer(ts, eigvals.reshape(-1))).reshape(
            ts.shape[0], niters, m)
        traces.append(n * (expeig * sqeigv1).sum(-1).mean(-1))
    subee = traces[0] - traces[1] / jnp.exp(ts)
    sub = -ts * n / jnp.exp(ts)
    return subee + sub


def _descriptor(x, ts, key):
    n = x.shape[0]
    G = _sparse_graph(x, _K)
    msid = _slq(G, n, _M, _NV, ts, key)
    return msid / n


def kernel(x_features, y_features):
    ts = jnp.asarray(np.logspace(-1, 1, 256), dtype=jnp.float32)
    mx = _descriptor(x_features, ts, jax.random.key(1))
    my = _descriptor(y_features, ts, jax.random.key(2))
    c = jnp.exp(-2.0 * (ts + 1.0 / ts))
    return jnp.amax(c * jnp.abs(mx - my))


# speedup vs baseline: 3.3686x; 1.8384x over previous
import functools

import numpy as np
import jax
import jax.numpy as jnp
from jax import lax
from jax.experimental import pallas as pl
from jax.experimental.pallas import tpu as pltpu
from jax.experimental.pallas import tpu_sc as plsc

_K = 5
_M = 10
_NV = 100
_TOL = 1e-05

_N = 5000
_D = 128
_BLK = 200
_NE = _K + 1


def _topk_body(x_ref, xt_ref, dd_ref, inds_ref):
    i = pl.program_id(0)
    G = jax.lax.dot_general(
        x_ref[...], xt_ref[...], (((1,), (0,)), ((), ())),
        preferred_element_type=jnp.float32)
    val = 2.0 * G - dd_ref[0:1, :]
    iota = jax.lax.broadcasted_iota(jnp.int32, val.shape, 1)
    rid = i * _BLK + jax.lax.broadcasted_iota(jnp.int32, (_BLK, 1), 0)
    for t in range(8):
        if t < _K + 1:
            m = jnp.max(val, axis=1, keepdims=True)
            idx = jnp.min(jnp.where(val == m, iota, jnp.int32(2**30)),
                          axis=1, keepdims=True)
            inds_ref[:, t:t + 1] = idx
            val = jnp.where(iota == idx, -jnp.inf, val)
        else:
            inds_ref[:, t:t + 1] = rid


def _knn_inds(x):
    n, d = x.shape
    dd = jnp.sum(x * x, axis=1)
    dd8 = jnp.broadcast_to(dd[None, :], (8, n))
    return pl.pallas_call(
        _topk_body,
        grid=(n // _BLK,),
        in_specs=[pl.BlockSpec((_BLK, d), lambda i: (i, 0)),
                  pl.BlockSpec((d, n), lambda i: (0, 0)),
                  pl.BlockSpec((8, n), lambda i: (0, 0))],
        out_specs=pl.BlockSpec((_BLK, 8), lambda i: (i, 0)),
        out_shape=jax.ShapeDtypeStruct((n, 8), jnp.int32),
    )(x, x.T, dd8)


def _adj_body(inds_ref, indsT_ref, a_ref, deg_ref):
    i = pl.program_id(0)
    coliota = jax.lax.broadcasted_iota(jnp.int32, (_BLK, _N), 1)
    rid = i * _BLK + jax.lax.broadcasted_iota(jnp.int32, (_BLK, 1), 0)
    cmp = jnp.zeros((_BLK, _N), jnp.bool_)
    for a in range(_NE):
        cmp = cmp | (coliota == inds_ref[:, a:a + 1])
    for a in range(_NE):
        cmp = cmp | (rid == indsT_ref[a:a + 1, :])
    cmp = cmp & (coliota != rid)
    af = cmp.astype(jnp.float32)
    a_ref[...] = af.astype(jnp.bfloat16)
    deg_ref[...] = jnp.sum(af, axis=1, keepdims=True)


def _build_adj(x):
    n = x.shape[0]
    inds = _knn_inds(x)
    indsT = inds[:, :_NE].T
    A, deg = pl.pallas_call(
        _adj_body,
        grid=(n // _BLK,),
        in_specs=[pl.BlockSpec((_BLK, 8), lambda i: (i, 0)),
                  pl.BlockSpec((_NE, n), lambda i: (0, 0))],
        out_specs=[pl.BlockSpec((_BLK, n), lambda i: (i, 0)),
                   pl.BlockSpec((_BLK, 1), lambda i: (i, 0))],
        out_shape=[jax.ShapeDtypeStruct((n, n), jnp.bfloat16),
                   jax.ShapeDtypeStruct((n, 1), jnp.float32)],
    )(inds, indsT)
    dsq = 1.0 / jnp.sqrt(deg[:, 0])
    return A, dsq


_MVB = 200


def _mv_body(a_ref, ys_ref, o_ref):
    af = a_ref[...].astype(jnp.float32)
    o_ref[...] = jax.lax.dot_general(
        af, ys_ref[...], (((1,), (0,)), ((), ())),
        preferred_element_type=jnp.float32)


def _matvec(A, dsq, S):
    n, nv = S.shape
    Y = dsq[:, None] * S
    P = pl.pallas_call(
        _mv_body,
        grid=(n // _MVB,),
        in_specs=[pl.BlockSpec((_MVB, n), lambda i: (i, 0)),
                  pl.BlockSpec((n, nv), lambda i: (0, 0))],
        out_specs=pl.BlockSpec((_MVB, nv), lambda i: (i, 0)),
        out_shape=jax.ShapeDtypeStruct((n, nv), jnp.float32),
    )(A, Y)
    return S - dsq[:, None] * P


def _lanczos(G, m, nv, key):
    A, dsq = G
    n = dsq.shape[0]
    SV = jax.random.normal(key, (n, nv), dtype=jnp.float32)
    SV = SV / jnp.linalg.norm(SV, axis=0)
    V = jnp.zeros((n, m, nv), dtype=jnp.float32)
    T = jnp.zeros((nv, m, m), dtype=jnp.float32)
    V = V.at[:, 0, :].set(SV)
    w = _matvec(A, dsq, SV)
    alpha = jnp.einsum('ij,ij->j', w, SV)
    w = w - alpha[None, :] * SV
    beta = jnp.sqrt(jnp.einsum('ij,ij->j', w, w))
    T = T.at[:, 0, 0].set(alpha)
    T = T.at[:, 0, 1].set(beta)
    T = T.at[:, 1, 0].set(beta)
    w = w / beta[None, :]
    V = V.at[:, 1, :].set(w)
    done = jnp.array(False)
    for i in range(1, m):
        SVold = V[:, i - 1, :]
        SVi = V[:, i, :]
        w = _matvec(A, dsq, SVi)
        w = w - beta[None, :] * SVold
        alpha = jnp.einsum('ij,ij->j', w, SVi)
        Tc = T.at[:, i, i].set(alpha)
        if i < m - 1:
            w = w - alpha[None, :] * SVi
            t = jnp.einsum('ijk,ik->jk', V, w)
            w = w - jnp.einsum('ijk,jk->ik', V, t)
            beta_new = jnp.sqrt(jnp.einsum('ij,ij->j', w, w))
            w = w / beta_new[None, :]
            Tc = Tc.at[:, i, i + 1].set(beta_new)
            Tc = Tc.at[:, i + 1, i].set(beta_new)
            innerprod = jnp.einsum('ijk,ik->jk', V, w)

            def cond_fn(carry):
                cnt, w_c, ip_c = carry
                return jnp.logical_and(cnt < 100, (ip_c > _TOL).sum() > 0)

            def body_fn(carry):
                cnt, w_c, ip_c = carry
                t_c = jnp.einsum('ijk,ik->jk', V, w_c)
                w_c = w_c - jnp.einsum('ijk,jk->ik', V, t_c)
                w_c = w_c / jnp.linalg.norm(w_c, axis=0)[None, :]
                ip_c = jnp.einsum('ijk,ik->jk', V, w_c)
                return (cnt + 1, w_c, ip_c)

            cnt, w, innerprod = jax.lax.while_loop(
                cond_fn, body_fn, (jnp.int32(0), w, innerprod))
            reortho = cnt < 100
            Vc = V.at[:, i + 1, :].set(w)
            T = jnp.where(done, T, Tc)
            V = jnp.where(done, V, Vc)
            beta = jnp.where(done, beta, beta_new)
            break_cond = jnp.logical_or(
                (jnp.abs(beta_new) > 1e-06).sum() == 0,
                jnp.logical_not(reortho))
            done = jnp.logical_or(done, break_cond)
        else:
            T = jnp.where(done, T, Tc)
    return T, V


def _slq(G, n, m, niters, ts, key):
    T, _ = _lanczos(G, m, niters, key)
    eigvals, eigvecs = jnp.linalg.eigh(T)
    sqeigv1 = eigvecs[:, 0, :] ** 2
    traces = []
    for f in (jnp.exp, lambda v: v):
        expeig = f(-jnp.outer(ts, eigvals.reshape(-1))).reshape(
            ts.shape[0], niters, m)
        traces.append(n * (expeig * sqeigv1).sum(-1).mean(-1))
    subee = traces[0] - traces[1] / jnp.exp(ts)
    sub = -ts * n / jnp.exp(ts)
    return subee + sub


def _descriptor(x, ts, key):
    n = x.shape[0]
    G = _build_adj(x)
    msid = _slq(G, n, _M, _NV, ts, key)
    return msid / n


def kernel(x_features, y_features):
    ts = jnp.asarray(np.logspace(-1, 1, 256), dtype=jnp.float32)
    mx = _descriptor(x_features, ts, jax.random.key(1))
    my = _descriptor(y_features, ts, jax.random.key(2))
    c = jnp.exp(-2.0 * (ts + 1.0 / ts))
    return jnp.amax(c * jnp.abs(mx - my))
